# baseline (speedup 1.0000x reference)
import jax
import jax.numpy as jnp
from jax import lax
from jax.experimental import pallas as pl
from jax.experimental.pallas import tpu as pltpu
from jax.experimental.pallas import tpu_sc as plsc

N = 100000
D = 128
NUM_SEGMENTS = 256
NC = 2
NS = 16
NW = NC * NS
SEG_PER_W = NUM_SEGMENTS // NW
BLK = 64
L = 16
NBLK_TOT = (N + BLK - 1) // BLK
SAMP_PAD = ((NBLK_TOT + 1 + L - 1) // L) * L

_GDN = lax.GatherDimensionNumbers(
    offset_dims=(), collapsed_slice_dims=(0,), start_index_map=(0,))


def _dg(v, idx):
    return lax.gather(v, idx.reshape(L, 1), _GDN, (1,),
                      mode=lax.GatherScatterMode.PROMISE_IN_BOUNDS)


def _lane_allsum(v):
    iota = lax.iota(jnp.int32, L)
    for sh in (8, 4, 2, 1):
        v = v + _dg(v, (iota + sh) & (L - 1))
    return v


def _sc_body(x_hbm, batch_hbm, w_hbm, b_hbm, samp_hbm, out_hbm,
             w_v, b_v, samp_v, xb_a, xb_b, idx_a, idx_b, acc_v,
             sem_a, sem_b):
    wid = lax.axis_index("s") * NC + lax.axis_index("c")

    pltpu.sync_copy(w_hbm, w_v)
    pltpu.sync_copy(b_hbm, b_v)
    pltpu.sync_copy(samp_hbm, samp_v)

    zv = jnp.zeros((L,), jnp.float32)
    for s in range(SEG_PER_W):
        for k in range(D // L):
            acc_v[s, pl.ds(k * L, L)] = zv

    ws = [-w_v[pl.ds(k * L, L)] for k in range(D // L)]
    bn = (-b_v[pl.ds(0, L)])[0]

    seg_lo = wid * SEG_PER_W
    seg_hi = seg_lo + SEG_PER_W

    cnt_a = jnp.zeros((L,), jnp.int32)
    cnt_b = jnp.zeros((L,), jnp.int32)
    one = jnp.ones((L,), jnp.int32)
    nil = jnp.zeros((L,), jnp.int32)
    for t in range(SAMP_PAD // L):
        sv = samp_v[pl.ds(t * L, L)]
        cnt_a = cnt_a + jnp.where(sv < seg_lo, one, nil)
        cnt_b = cnt_b + jnp.where(sv < seg_hi, one, nil)
    cnt_a = _lane_allsum(cnt_a)
    cnt_b = _lane_allsum(cnt_b)
    first_samp = samp_v[pl.ds(0, L)][0]
    pstart = cnt_a[0] - jnp.where(first_samp < seg_lo, 1, 0)
    nblk = cnt_b[0] - pstart

    bufs = ((xb_a, idx_a, sem_a), (xb_b, idx_b, sem_b))

    def blk_row0(i):
        p = (pstart + i) * BLK
        return p, jnp.minimum(p, N - BLK)

    def start_blk(i, xb_v, idx_v, sem):
        _, bs = blk_row0(i)
        pltpu.async_copy(x_hbm.at[pl.ds(bs, BLK)], xb_v, sem)
        pltpu.async_copy(batch_hbm.at[pl.ds(bs, BLK)],
                         idx_v.at[pl.ds(0, BLK)], sem)

    def wait_blk(xb_v, idx_v, sem):
        pltpu.make_async_copy(x_hbm.at[pl.ds(0, BLK)], xb_v, sem).wait()
        pltpu.make_async_copy(batch_hbm.at[pl.ds(0, BLK)],
                              idx_v.at[pl.ds(0, BLK)], sem).wait()

    def compute_blk(i, xb_v, idx_v):
        p, bs = blk_row0(i)
        dup = p - bs

        @plsc.parallel_loop(0, BLK, 2, unroll=3)
        def _rows(r0):
            iv = idx_v[pl.ds(r0, L)]
            iota = lax.iota(jnp.int32, L)
            mlo = iota < 8
            sa, sb = iv[0], iv[1]
            seg_a = (sa - seg_lo) & (SEG_PER_W - 1)
            seg_b = (sb - seg_lo) & (SEG_PER_W - 1)
            ok_a = (sa >= seg_lo) & (sa < seg_hi) & (r0 >= dup)
            ok_b = (sb >= seg_lo) & (sb < seg_hi) & (r0 + 1 >= dup)
            pen_a = jnp.where(ok_a, bn, 1e30)
            pen_b = jnp.where(ok_b, bn, 1e30)
            xa = [xb_v[r0, pl.ds(k * L, L)] for k in range(D // L)]
            xb = [xb_v[r0 + 1, pl.ds(k * L, L)] for k in range(D // L)]

            def partial(xs):
                m = [xs[k] * ws[k] for k in range(D // L)]
                return ((m[0] + m[1]) + (m[2] + m[3])) + \
                       ((m[4] + m[5]) + (m[6] + m[7]))

            pa, pb = partial(xa), partial(xb)
            a1 = pa + _dg(pa, iota ^ 8)
            b1 = pb + _dg(pb, iota ^ 8)
            c = jnp.where(mlo, a1, b1)
            for sh in (4, 2, 1):
                c = c + _dg(c, iota ^ sh)
            u = c + jnp.where(mlo, pen_a, pen_b)
            wt2 = 1.0 / (1.0 + jnp.exp(u))
            wt_a, wt_b = wt2[0], wt2[8]
            for k in range(D // L):
                plsc.addupdate(acc_v.at[seg_a, pl.ds(k * L, L)], xa[k] * wt_a)
            for k in range(D // L):
                plsc.addupdate(acc_v.at[seg_b, pl.ds(k * L, L)], xb[k] * wt_b)

    @pl.when(nblk > 0)
    def _():
        start_blk(0, *bufs[0])

    def blk_pair(i, carry):
        for b in range(2):
            blk = 2 * i + b

            @pl.when(blk < nblk)
            def _(blk=blk, b=b):
                @pl.when(blk + 1 < nblk)
                def _():
                    start_blk(blk + 1, *bufs[1 - b])

                wait_blk(*bufs[b])
                compute_blk(blk, bufs[b][0], bufs[b][1])

        return carry

    lax.fori_loop(0, (nblk + 1) // 2, blk_pair, 0, unroll=False)

    pltpu.sync_copy(acc_v, out_hbm.at[pl.ds(seg_lo, SEG_PER_W)])


@jax.jit
def kernel(x, batch, W, b):
    batch_i = batch.astype(jnp.int32)
    samp = batch_i[::BLK]
    samp = jnp.pad(samp, (0, SAMP_PAD - samp.shape[0]),
                   constant_values=NUM_SEGMENTS)
    w_flat = W.reshape(D).astype(jnp.float32)
    b_pad = jnp.pad(b.astype(jnp.float32), (0, L - 1))

    mesh = plsc.VectorSubcoreMesh(core_axis_name="c", subcore_axis_name="s",
                                  num_cores=NC, num_subcores=NS)
    run = pl.kernel(
        _sc_body,
        out_type=jax.ShapeDtypeStruct((NUM_SEGMENTS, D), jnp.float32),
        mesh=mesh,
        scratch_types=[
            pltpu.VMEM((D,), jnp.float32),
            pltpu.VMEM((L,), jnp.float32),
            pltpu.VMEM((SAMP_PAD,), jnp.int32),
            pltpu.VMEM((BLK, D), jnp.float32),
            pltpu.VMEM((BLK, D), jnp.float32),
            pltpu.VMEM((BLK + L,), jnp.int32),
            pltpu.VMEM((BLK + L,), jnp.int32),
            pltpu.VMEM((SEG_PER_W, D), jnp.float32),
            pltpu.SemaphoreType.DMA,
            pltpu.SemaphoreType.DMA,
        ],
    )
    return run(x, batch_i, w_flat, b_pad, samp)


# speedup vs baseline: 1.1780x; 1.1780x over previous
import jax
import jax.numpy as jnp
from jax import lax
from jax.experimental import pallas as pl
from jax.experimental.pallas import tpu as pltpu
from jax.experimental.pallas import tpu_sc as plsc

N = 100000
D = 128
NUM_SEGMENTS = 256
NC = 2
NS = 16
NW = NC * NS
SEG_PER_W = NUM_SEGMENTS // NW
BLK = 64
L = 16
NBLK_TOT = (N + BLK - 1) // BLK
SAMP_PAD = ((NBLK_TOT + 1 + L - 1) // L) * L

_GDN = lax.GatherDimensionNumbers(
    offset_dims=(), collapsed_slice_dims=(0,), start_index_map=(0,))


def _dg(v, idx):
    return lax.gather(v, idx.reshape(L, 1), _GDN, (1,),
                      mode=lax.GatherScatterMode.PROMISE_IN_BOUNDS)


def _lane_allsum(v):
    iota = lax.iota(jnp.int32, L)
    for sh in (8, 4, 2, 1):
        v = v + _dg(v, (iota + sh) & (L - 1))
    return v


def _sc_body(x_hbm, batch_hbm, w_hbm, b_hbm, samp_hbm, out_hbm,
             w_v, b_v, samp_v, xb_a, xb_b, idx_a, idx_b, acc_v,
             sem_a, sem_b):
    wid = lax.axis_index("s") * NC + lax.axis_index("c")

    pltpu.sync_copy(w_hbm, w_v)
    pltpu.sync_copy(b_hbm, b_v)
    pltpu.sync_copy(samp_hbm, samp_v)

    zv = jnp.zeros((L,), jnp.float32)
    for s in range(SEG_PER_W):
        for k in range(D // L):
            acc_v[s, pl.ds(k * L, L)] = zv

    ws = [-w_v[pl.ds(k * L, L)] for k in range(D // L)]
    bn = (-b_v[pl.ds(0, L)])[0]

    seg_lo = wid * SEG_PER_W
    seg_hi = seg_lo + SEG_PER_W

    cnt_a = jnp.zeros((L,), jnp.int32)
    cnt_b = jnp.zeros((L,), jnp.int32)
    one = jnp.ones((L,), jnp.int32)
    nil = jnp.zeros((L,), jnp.int32)
    for t in range(SAMP_PAD // L):
        sv = samp_v[pl.ds(t * L, L)]
        cnt_a = cnt_a + jnp.where(sv < seg_lo, one, nil)
        cnt_b = cnt_b + jnp.where(sv < seg_hi, one, nil)
    cnt_a = _lane_allsum(cnt_a)
    cnt_b = _lane_allsum(cnt_b)
    first_samp = samp_v[pl.ds(0, L)][0]
    pstart = cnt_a[0] - jnp.where(first_samp < seg_lo, 1, 0)
    nblk = cnt_b[0] - pstart

    bufs = ((xb_a, idx_a, sem_a), (xb_b, idx_b, sem_b))

    def blk_row0(i):
        p = (pstart + i) * BLK
        return p, jnp.minimum(p, N - BLK)

    def start_blk(i, xb_v, idx_v, sem):
        _, bs = blk_row0(i)
        pltpu.async_copy(x_hbm.at[pl.ds(bs, BLK)], xb_v, sem)
        pltpu.async_copy(batch_hbm.at[pl.ds(bs, BLK)],
                         idx_v.at[pl.ds(0, BLK)], sem)

    def wait_blk(xb_v, idx_v, sem):
        pltpu.make_async_copy(x_hbm.at[pl.ds(0, BLK)], xb_v, sem).wait()
        pltpu.make_async_copy(batch_hbm.at[pl.ds(0, BLK)],
                              idx_v.at[pl.ds(0, BLK)], sem).wait()

    def compute_blk(i, xb_v, idx_v):
        p, bs = blk_row0(i)
        dup = p - bs

        @plsc.parallel_loop(0, BLK, 2, unroll=2)
        def _rows(r0):
            iv = idx_v[pl.ds(r0, L)]
            iota = lax.iota(jnp.int32, L)
            mlo = iota < 8
            sa, sb = iv[0], iv[1]
            seg_a = (sa - seg_lo) & (SEG_PER_W - 1)
            seg_b = (sb - seg_lo) & (SEG_PER_W - 1)
            ok_a = (sa >= seg_lo) & (sa < seg_hi) & (r0 >= dup)
            ok_b = (sb >= seg_lo) & (sb < seg_hi) & (r0 + 1 >= dup)
            pen_a = jnp.where(ok_a, bn, 1e30)
            pen_b = jnp.where(ok_b, bn, 1e30)
            xa = [xb_v[r0, pl.ds(k * L, L)] for k in range(D // L)]
            xb = [xb_v[r0 + 1, pl.ds(k * L, L)] for k in range(D // L)]

            def partial(xs):
                m = [xs[k] * ws[k] for k in range(D // L)]
                return ((m[0] + m[1]) + (m[2] + m[3])) + \
                       ((m[4] + m[5]) + (m[6] + m[7]))

            pa, pb = partial(xa), partial(xb)
            a1 = pa + _dg(pa, iota ^ 8)
            b1 = pb + _dg(pb, iota ^ 8)
            c = jnp.where(mlo, a1, b1)
            for sh in (4, 2, 1):
                c = c + _dg(c, iota ^ sh)
            u = c + jnp.where(mlo, pen_a, pen_b)
            wt2 = 1.0 / (1.0 + jnp.exp(u))
            wt_a, wt_b = wt2[0], wt2[8]
            for k in range(D // L):
                plsc.addupdate(acc_v.at[seg_a, pl.ds(k * L, L)], xa[k] * wt_a)
            for k in range(D // L):
                plsc.addupdate(acc_v.at[seg_b, pl.ds(k * L, L)], xb[k] * wt_b)

    @pl.when(nblk > 0)
    def _():
        start_blk(0, *bufs[0])

    def blk_pair(i, carry):
        for b in range(2):
            blk = 2 * i + b

            @pl.when(blk < nblk)
            def _(blk=blk, b=b):
                @pl.when(blk + 1 < nblk)
                def _():
                    start_blk(blk + 1, *bufs[1 - b])

                wait_blk(*bufs[b])
                compute_blk(blk, bufs[b][0], bufs[b][1])

        return carry

    lax.fori_loop(0, (nblk + 1) // 2, blk_pair, 0, unroll=False)

    pltpu.sync_copy(acc_v, out_hbm.at[pl.ds(seg_lo, SEG_PER_W)])


@jax.jit
def kernel(x, batch, W, b):
    batch_i = batch.astype(jnp.int32)
    samp = batch_i[::BLK]
    samp = jnp.pad(samp, (0, SAMP_PAD - samp.shape[0]),
                   constant_values=NUM_SEGMENTS)
    w_flat = W.reshape(D).astype(jnp.float32)
    b_pad = jnp.pad(b.astype(jnp.float32), (0, L - 1))

    mesh = plsc.VectorSubcoreMesh(core_axis_name="c", subcore_axis_name="s",
                                  num_cores=NC, num_subcores=NS)
    run = pl.kernel(
        _sc_body,
        out_type=jax.ShapeDtypeStruct((NUM_SEGMENTS, D), jnp.float32),
        mesh=mesh,
        scratch_types=[
            pltpu.VMEM((D,), jnp.float32),
            pltpu.VMEM((L,), jnp.float32),
            pltpu.VMEM((SAMP_PAD,), jnp.int32),
            pltpu.VMEM((BLK, D), jnp.float32),
            pltpu.VMEM((BLK, D), jnp.float32),
            pltpu.VMEM((BLK + L,), jnp.int32),
            pltpu.VMEM((BLK + L,), jnp.int32),
            pltpu.VMEM((SEG_PER_W, D), jnp.float32),
            pltpu.SemaphoreType.DMA,
            pltpu.SemaphoreType.DMA,
        ],
    )
    return run(x, batch_i, w_flat, b_pad, samp)
